# Initial kernel scaffold; baseline (speedup 1.0000x reference)
import jax
import jax.numpy as jnp
from jax.experimental import pallas as pl


def kernel(z, embedding):
    raise NotImplementedError("write your pallas kernel here")


# speedup vs baseline: 1.3876x; 1.3876x over previous
import functools

import jax
import jax.numpy as jnp
from jax import lax
from jax.experimental import pallas as pl
from jax.experimental.pallas import tpu as pltpu
from jax.experimental.pallas import tpu_sc as plsc

N_E = 8192
E_DIM = 32
BETA = 0.25
N_TOK = 32 * 1024
BLK = 512
NBLK = N_TOK // BLK

_EPS = 1e-12


def _vq_tc_body(z_ref, emb_ref, idx_ref, en_ref, loss_ref, acc_ref,
                enn2_ref):
    step = pl.program_id(0)

    @pl.when(step == 0)
    def _():
        emb = emb_ref[...]
        en0 = emb * lax.rsqrt(jnp.sum(emb * emb, axis=1, keepdims=True) + _EPS)
        en_ref[...] = en0
        enn2_ref[...] = jnp.reshape(jnp.sum(en0 * en0, axis=1), (1, N_E))
        acc_ref[0] = 0.0

    en = en_ref[...]

    z = z_ref[...]
    zn = z * lax.rsqrt(jnp.sum(z * z, axis=1, keepdims=True) + _EPS)

    s = lax.dot_general(zn, en, (((1,), (1,)), ((), ())),
                        preferred_element_type=jnp.float32)
    znn2 = jnp.sum(zn * zn, axis=1, keepdims=True)
    d = (znn2 + enn2_ref[...]) - 2.0 * s

    dmin = jnp.min(d, axis=1, keepdims=True)
    iota = lax.broadcasted_iota(jnp.int32, (BLK, N_E), 1)
    cand = jnp.where(d == dmin, iota, N_E)
    idx_ref[0, 0, :] = jnp.min(cand, axis=1)

    acc_ref[0] += jnp.sum(dmin)

    @pl.when(step == NBLK - 1)
    def _():
        m = acc_ref[0] / float(N_TOK * E_DIM)
        loss_ref[0] = BETA * m + m


_SC_INFO = plsc.get_sparse_core_info()
_NW = _SC_INFO.num_cores * _SC_INFO.num_subcores
_BPW = N_TOK // _NW
_CH = 128
_NCH = _BPW // _CH

_sc_mesh = plsc.VectorSubcoreMesh(core_axis_name="c", subcore_axis_name="s")


@functools.partial(
    pl.kernel,
    mesh=_sc_mesh,
    compiler_params=pltpu.CompilerParams(use_tc_tiling_on_sc=False),
    out_type=jax.ShapeDtypeStruct((N_TOK, E_DIM), jnp.float32),
    scratch_types=[
        pltpu.VMEM((_NCH, _CH), jnp.int32),
        pltpu.VMEM((_BPW, E_DIM), jnp.float32),
        pltpu.SemaphoreType.DMA,
    ],
)
def _sc_gather(en_hbm, idx_hbm, out_hbm, idx_v, rows_v, sem):
    wid = lax.axis_index("s") * _SC_INFO.num_cores + lax.axis_index("c")
    pltpu.sync_copy(idx_hbm.at[wid], idx_v)
    copies = []
    for j in range(_NCH):
        copies.append(
            pltpu.async_copy(en_hbm.at[idx_v.at[j]],
                             rows_v.at[pl.ds(j * _CH, _CH)], sem))
    for c in copies:
        c.wait()
    pltpu.sync_copy(rows_v, out_hbm.at[pl.ds(wid * _BPW, _BPW)])


def kernel(z, embedding):
    zf = jnp.reshape(z, (N_TOK, E_DIM))

    idx3, en, loss1 = pl.pallas_call(
        _vq_tc_body,
        grid=(NBLK,),
        in_specs=[
            pl.BlockSpec((BLK, E_DIM), lambda i: (i, 0)),
            pl.BlockSpec((N_E, E_DIM), lambda i: (0, 0)),
        ],
        out_specs=[
            pl.BlockSpec((1, 1, BLK), lambda i: (i, 0, 0)),
            pl.BlockSpec((N_E, E_DIM), lambda i: (0, 0)),
            pl.BlockSpec(memory_space=pltpu.SMEM),
        ],
        out_shape=[
            jax.ShapeDtypeStruct((NBLK, 1, BLK), jnp.int32),
            jax.ShapeDtypeStruct((N_E, E_DIM), jnp.float32),
            jax.ShapeDtypeStruct((1,), jnp.float32),
        ],
        scratch_shapes=[pltpu.SMEM((1,), jnp.float32),
                        pltpu.VMEM((1, N_E), jnp.float32)],
    )(zf, embedding)

    idx_flat = jnp.reshape(idx3, (N_TOK,))
    z_q = _sc_gather(en, jnp.reshape(idx_flat, (_NW, _NCH, _CH)))
    return (jnp.reshape(z_q, z.shape), loss1[0],
            jnp.reshape(idx_flat, z.shape[:-1]))
